# baseline (speedup 1.0000x reference)
import functools

import jax
import jax.numpy as jnp
from jax import lax
from jax.experimental import pallas as pl
from jax.experimental.pallas import tpu as pltpu
from jax.experimental.pallas import tpu_sc as plsc

_D = 1024
_S = 8192
_B = 4
_NC = 2
_NS = 16
_NW = _NC * _NS
_RPW = _S // _NW
_CHUNK = 16
_NCHUNK = _RPW // _CHUNK
_CW = _CHUNK * _D
_VL = 16


def _body(x_hbm, cls_hbm, pos_hbm, out_hbm, pos_v, x_v, cls_v):
    cid = lax.axis_index("c")
    sid = lax.axis_index("s")
    wid = sid * _NC + cid
    base = wid * _RPW

    @pl.when(wid < _B)
    def _():
        pltpu.sync_copy(cls_hbm, cls_v)
        pltpu.sync_copy(pos_hbm.at[pl.ds(0, _D)], pos_v.at[pl.ds(0, _D)])

        def cls_add(k, carry):
            ds = pl.ds(k * _VL, _VL)
            plsc.addupdate(cls_v.at[ds], pos_v[ds])
            return carry

        lax.fori_loop(0, _D // _VL, cls_add, 0)
        pltpu.sync_copy(cls_v, out_hbm.at[wid, pl.ds(0, _D)])

    def chunk_body(ci, carry):
        off = (base + ci * _CHUNK) * _D
        pltpu.sync_copy(pos_hbm.at[pl.ds(_D + off, _CW)], pos_v)
        for b in range(_B):
            pltpu.sync_copy(x_hbm.at[b, pl.ds(off, _CW)], x_v)

            def add_body(k, c):
                ds = pl.ds(k * _VL, _VL)
                plsc.addupdate(x_v.at[ds], pos_v[ds])
                return c

            lax.fori_loop(0, _CW // _VL, add_body, 0)
            pltpu.sync_copy(x_v, out_hbm.at[b, pl.ds(_D + off, _CW)])
        return carry

    lax.fori_loop(0, _NCHUNK, chunk_body, 0)


_pe_call = functools.partial(
    pl.kernel,
    out_type=jax.ShapeDtypeStruct((_B, (_S + 1) * _D), jnp.float32),
    mesh=plsc.VectorSubcoreMesh(core_axis_name="c", subcore_axis_name="s"),
    scratch_types=[
        pltpu.VMEM((_CW,), jnp.float32),
        pltpu.VMEM((_CW,), jnp.float32),
        pltpu.VMEM((_D,), jnp.float32),
    ],
)(_body)


@jax.jit
def kernel(x, cls_token, pos_table):
    xf = x.reshape(_B, _S * _D)
    clsf = cls_token.reshape(_D)
    posf = pos_table.reshape((_S + 1) * _D)
    out = _pe_call(xf, clsf, posf)
    return out.reshape(_B, _S + 1, _D)


# speedup vs baseline: 1.6953x; 1.6953x over previous
import functools

import jax
import jax.numpy as jnp
from jax import lax
from jax.experimental import pallas as pl
from jax.experimental.pallas import tpu as pltpu
from jax.experimental.pallas import tpu_sc as plsc

_D = 1024
_S = 8192
_B = 4
_NC = 2
_NS = 16
_NW = _NC * _NS
_RPW = _S // _NW
_CHUNK = 16
_NCHUNK = _RPW // _CHUNK
_CW = _CHUNK * _D
_VL = 16


def _body(x_hbm, cls_hbm, pos_hbm, out_hbm,
          x0, x1, p0, p1, cls_v, sx0, sx1, so0, so1, sp0, sp1):
    cid = lax.axis_index("c")
    sid = lax.axis_index("s")
    wid = sid * _NC + cid
    base = wid * _RPW

    X = (x0, x1)
    P = (p0, p1)
    SX = (sx0, sx1)
    SO = (so0, so1)
    SP = (sp0, sp1)

    def xs(cc, b):
        return x_hbm.at[b, pl.ds((base + cc * _CHUNK) * _D, _CW)]

    def os_(cc, b):
        return out_hbm.at[b, pl.ds((base + cc * _CHUNK) * _D + _D, _CW)]

    def ps(cc):
        return pos_hbm.at[pl.ds((base + cc * _CHUNK) * _D + _D, _CW)]

    @pl.when(wid < _B)
    def _():
        pltpu.sync_copy(cls_hbm, cls_v)
        pltpu.sync_copy(pos_hbm.at[pl.ds(0, _D)], p0.at[pl.ds(0, _D)])

        @plsc.parallel_loop(0, _D, step=_VL, unroll=8)
        def _(k):
            ds = pl.ds(k, _VL)
            plsc.addupdate(cls_v.at[ds], p0[ds])

        pltpu.sync_copy(cls_v, out_hbm.at[wid, pl.ds(0, _D)])

    def emit_item(cc, b, cpar, first_item=False, last_chunk=False,
                  last_item=False):
        xp = b % 2
        nxp = 1 - xp
        if b == 0:
            pltpu.make_async_copy(ps(cc), P[cpar], SP[cpar]).wait()
            if not last_chunk:
                pltpu.async_copy(ps(cc + 1), P[1 - cpar], SP[1 - cpar])
        pltpu.make_async_copy(xs(cc, b), X[xp], SX[xp]).wait()

        @plsc.parallel_loop(0, _CW, step=_VL, unroll=8)
        def _(k):
            ds = pl.ds(k, _VL)
            plsc.addupdate(X[xp].at[ds], P[cpar][ds])

        pltpu.async_copy(X[xp], os_(cc, b), SO[xp])
        if not last_item:
            if not first_item:
                pltpu.make_async_copy(X[nxp], os_(cc, b), SO[nxp]).wait()
            if b < _B - 1:
                pltpu.async_copy(xs(cc, b + 1), X[nxp], SX[nxp])
            else:
                pltpu.async_copy(xs(cc + 1, 0), X[nxp], SX[nxp])

    pltpu.async_copy(ps(0), p0, sp0)
    pltpu.async_copy(xs(0, 0), x0, sx0)
    for b in range(_B):
        emit_item(0, b, 0, first_item=(b == 0))
    for b in range(_B):
        emit_item(1, b, 1)

    def pair_body(j, carry):
        c0 = 2 * j
        for k in range(2):
            for b in range(_B):
                emit_item(c0 + k, b, k)
        return carry

    lax.fori_loop(1, _NCHUNK // 2 - 1, pair_body, 0)

    for b in range(_B):
        emit_item(_NCHUNK - 2, b, 0)
    for b in range(_B):
        emit_item(_NCHUNK - 1, b, 1, last_chunk=True, last_item=(b == _B - 1))

    pltpu.make_async_copy(x0, os_(_NCHUNK - 1, 2), so0).wait()
    pltpu.make_async_copy(x1, os_(_NCHUNK - 1, 3), so1).wait()


_pe_call = functools.partial(
    pl.kernel,
    out_type=jax.ShapeDtypeStruct((_B, (_S + 1) * _D), jnp.float32),
    mesh=plsc.VectorSubcoreMesh(core_axis_name="c", subcore_axis_name="s"),
    scratch_types=[
        pltpu.VMEM((_CW,), jnp.float32),
        pltpu.VMEM((_CW,), jnp.float32),
        pltpu.VMEM((_CW,), jnp.float32),
        pltpu.VMEM((_CW,), jnp.float32),
        pltpu.VMEM((_D,), jnp.float32),
        pltpu.SemaphoreType.DMA,
        pltpu.SemaphoreType.DMA,
        pltpu.SemaphoreType.DMA,
        pltpu.SemaphoreType.DMA,
        pltpu.SemaphoreType.DMA,
        pltpu.SemaphoreType.DMA,
    ],
)(_body)


@jax.jit
def kernel(x, cls_token, pos_table):
    xf = x.reshape(_B, _S * _D)
    clsf = cls_token.reshape(_D)
    posf = pos_table.reshape((_S + 1) * _D)
    out = _pe_call(xf, clsf, posf)
    return out.reshape(_B, _S + 1, _D)
